# baseline (speedup 1.0000x reference)
import jax
import jax.numpy as jnp
from jax.experimental import pallas as pl

_CODES = 512
_LB = 1024


def _vq_body(x_ref, c_ref, o_ref):
    xb = x_ref[0, 0]
    cb = c_ref[0]
    xc2 = jax.lax.dot_general(
        xb * 2.0, cb,
        dimension_numbers=(((1,), (1,)), ((), ())),
        preferred_element_type=jnp.float32,
    )
    x2 = jnp.sum(xb * xb, axis=1, keepdims=True)
    c2 = jnp.sum(cb * cb, axis=1)[None, :]
    dist = (x2 - xc2) + c2
    minval = jnp.min(dist, axis=1, keepdims=True)
    iota = jax.lax.broadcasted_iota(jnp.int32, dist.shape, 1)
    first = jnp.min(jnp.where(dist == minval, iota, _CODES),
                    axis=1, keepdims=True)
    o_ref[0, 0] = (iota == first).astype(jnp.float32)


def kernel(x, c):
    b, h, l, d = x.shape
    s = c.shape[1]
    grid = (b, h, l // _LB)
    out = pl.pallas_call(
        _vq_body,
        grid=grid,
        in_specs=[
            pl.BlockSpec((1, 1, _LB, d), lambda i, j, k: (i, j, k, 0)),
            pl.BlockSpec((1, s, d), lambda i, j, k: (j, 0, 0)),
        ],
        out_specs=pl.BlockSpec((1, 1, _LB, s), lambda i, j, k: (i, j, k, 0)),
        out_shape=jax.ShapeDtypeStruct((b, h, l, s), jnp.float32),
    )(x, c)
    return (out, c)


# speedup vs baseline: 1.2050x; 1.2050x over previous
import jax
import jax.numpy as jnp
from jax.experimental import pallas as pl

_CODES = 512
_LB = 4096


def _vq_body(x_ref, c_ref, o_ref):
    xb = x_ref[0, 0]
    cb = c_ref[0]
    xc2 = jax.lax.dot_general(
        xb * 2.0, cb,
        dimension_numbers=(((1,), (1,)), ((), ())),
        preferred_element_type=jnp.float32,
    )
    x2 = jnp.sum(xb * xb, axis=1, keepdims=True)
    c2 = jnp.sum(cb * cb, axis=1)[None, :]
    dist = (x2 - xc2) + c2
    first = jnp.argmin(dist, axis=1)[:, None]
    iota = jax.lax.broadcasted_iota(jnp.int32, dist.shape, 1)
    o_ref[0, 0] = (iota == first).astype(jnp.float32)


def kernel(x, c):
    b, h, l, d = x.shape
    s = c.shape[1]
    grid = (b, h, l // _LB)
    out = pl.pallas_call(
        _vq_body,
        grid=grid,
        in_specs=[
            pl.BlockSpec((1, 1, _LB, d), lambda i, j, k: (i, j, k, 0)),
            pl.BlockSpec((1, s, d), lambda i, j, k: (j, 0, 0)),
        ],
        out_specs=pl.BlockSpec((1, 1, _LB, s), lambda i, j, k: (i, j, k, 0)),
        out_shape=jax.ShapeDtypeStruct((b, h, l, s), jnp.float32),
    )(x, c)
    return (out, c)
